# baseline (speedup 1.0000x reference)
import functools

import jax
import jax.numpy as jnp
from jax import lax
from jax.experimental import pallas as pl
from jax.experimental.pallas import tpu as pltpu
from jax.experimental.pallas import tpu_sc as plsc

_SC_ROWS = 32
_TC_BLOCK_ROWS = 32
_NCHUNKS = 8


def _tc_kernel(x_ref, t_ref, acc1_ref, acc5_ref, *, scale, nchunks):
    i = pl.program_id(0)
    r, n = x_ref.shape
    cw = n // nchunks
    t = t_ref[pl.ds(i * r, r), :]
    maxes = []
    for c in range(nchunks):
        xc = x_ref[:, c * cw:(c + 1) * cw]
        colc = lax.broadcasted_iota(jnp.int32, (r, cw), 1) + c * cw
        maxes.append(jnp.max(jnp.where(colc == t, xc, -jnp.inf),
                             axis=1, keepdims=True))
    vt = functools.reduce(jnp.maximum, maxes)
    cnts = []
    for c in range(nchunks):
        xc = x_ref[:, c * cw:(c + 1) * cw]
        colc = lax.broadcasted_iota(jnp.int32, (r, cw), 1) + c * cw
        pred = (xc > vt) | ((xc == vt) & (colc < t))
        cnts.append(jnp.sum(pred.astype(jnp.float32), axis=1, keepdims=True))
    rank = functools.reduce(jnp.add, cnts)
    a1 = jnp.sum((rank < 1.0).astype(jnp.float32)).reshape(1, 1) * scale
    a5 = jnp.sum((rank < 5.0).astype(jnp.float32)).reshape(1, 1) * scale

    @pl.when(i == 0)
    def _init():
        acc1_ref[...] = a1
        acc5_ref[...] = a5

    @pl.when(i != 0)
    def _accum():
        acc1_ref[...] += a1
        acc5_ref[...] += a5


def _make_sc_kernel(b, n, sc_rows, row_base):
    info = plsc.get_sparse_core_info()
    nc, ns, nl = info.num_cores, info.num_subcores, info.num_lanes
    nvec = n // nl
    unroll = 8

    @functools.partial(
        pl.kernel,
        mesh=plsc.VectorSubcoreMesh(core_axis_name="c", subcore_axis_name="s"),
        out_type=jax.ShapeDtypeStruct((sc_rows, nl), jnp.float32),
        scratch_types=[
            pltpu.VMEM((n,), jnp.float32),
            pltpu.VMEM((nl,), jnp.int32),
            pltpu.VMEM((nl,), jnp.float32),
            pltpu.VMEM((2 * nl,), jnp.float32),
        ],
    )
    def sc_count(x_hbm, t_hbm, out_hbm, xrow_v, t_v, res_v, pad_v):
        wid = lax.axis_index("s") * nc + lax.axis_index("c")

        @pl.when(wid < sc_rows)
        def _():
            row = row_base + wid
            pltpu.sync_copy(x_hbm.at[row], xrow_v)
            pltpu.sync_copy(t_hbm.at[wid], t_v)
            tsplat = t_v[...]
            iota = lax.iota(jnp.int32, nl)
            t_scalar = tsplat[0]
            lane = t_scalar % nl
            win = xrow_v[pl.ds(t_scalar - lane, nl)]
            pad_v[pl.ds(0, nl)] = win
            pad_v[pl.ds(nl, nl)] = win
            vt_scalar = pad_v[pl.ds(lane, nl)][0]
            vt = jnp.full((nl,), vt_scalar, jnp.float32)
            one = jnp.full((nl,), 1.0, jnp.float32)
            zero = jnp.full((nl,), 0.0, jnp.float32)

            def body(k, acc):
                base = k * (nl * unroll)
                for u in range(unroll):
                    xv = xrow_v[pl.ds(base + u * nl, nl)]
                    col = iota + (base + u * nl)
                    pred = (xv > vt) | ((xv == vt) & (col < tsplat))
                    acc = acc + jnp.where(pred, one, zero)
                return acc

            acc = lax.fori_loop(0, nvec // unroll, body, zero)
            rank = acc[0]
            for lane in range(1, nl):
                rank = rank + acc[lane]
            res_v[...] = jnp.full((nl,), rank, jnp.float32)
            pltpu.sync_copy(res_v, out_hbm.at[wid])

    return sc_count


@jax.jit
def kernel(outputs, targets):
    b, n = outputs.shape
    scale = 100.0 / b
    tc_rows = b - _SC_ROWS
    r = _TC_BLOCK_ROWS
    t2 = targets.astype(jnp.int32)
    t2d = t2.reshape(b, 1)
    body = functools.partial(_tc_kernel, scale=scale, nchunks=_NCHUNKS)
    a1, a5 = pl.pallas_call(
        body,
        grid=(tc_rows // r,),
        in_specs=[
            pl.BlockSpec((r, n), lambda i: (i, 0)),
            pl.BlockSpec((b, 1), lambda i: (0, 0)),
        ],
        out_specs=[
            pl.BlockSpec((1, 1), lambda i: (0, 0)),
            pl.BlockSpec((1, 1), lambda i: (0, 0)),
        ],
        out_shape=[
            jax.ShapeDtypeStruct((1, 1), jnp.float32),
            jax.ShapeDtypeStruct((1, 1), jnp.float32),
        ],
    )(outputs, t2d)

    sc_count = _make_sc_kernel(b, n, _SC_ROWS, tc_rows)
    sc_t = jnp.broadcast_to(t2[tc_rows:, None], (_SC_ROWS, 16))
    sc_ranks = sc_count(outputs, sc_t)[:, 0]
    sc1 = jnp.sum((sc_ranks < 1.0).astype(jnp.float32)) * scale
    sc5 = jnp.sum((sc_ranks < 5.0).astype(jnp.float32)) * scale
    out1 = a1.reshape(1) + sc1
    out5 = a5.reshape(1) + sc5
    return (out1, out5)


# speedup vs baseline: 2.7195x; 2.7195x over previous
import functools

import jax
import jax.numpy as jnp
from jax.experimental import pallas as pl

_BANDS = 4
_ROWS_PER_STEP = 8
_NCHUNKS = 4


def _band_rank(x, t, nchunks):
    r, n = x.shape
    cw = n // nchunks
    maxes = []
    for c in range(nchunks):
        xc = x[:, c * cw:(c + 1) * cw]
        colc = jax.lax.broadcasted_iota(jnp.int32, (r, cw), 1) + c * cw
        maxes.append(jnp.max(jnp.where(colc == t, xc, -jnp.inf),
                             axis=1, keepdims=True))
    vt = functools.reduce(jnp.maximum, maxes)
    cnts = []
    for c in range(nchunks):
        xc = x[:, c * cw:(c + 1) * cw]
        colc = jax.lax.broadcasted_iota(jnp.int32, (r, cw), 1) + c * cw
        pred = (xc > vt) | ((xc == vt) & (colc < t))
        cnts.append(jnp.sum(pred.astype(jnp.float32), axis=1, keepdims=True))
    return functools.reduce(jnp.add, cnts)


def _acc_kernel(*refs, scale, nchunks, bands, band_rows):
    x_refs = refs[:bands]
    t_ref = refs[bands]
    acc1_ref, acc5_ref = refs[bands + 1], refs[bands + 2]
    i = pl.program_id(0)
    r = x_refs[0].shape[1]
    a1 = jnp.zeros((1, 1), jnp.float32)
    a5 = jnp.zeros((1, 1), jnp.float32)
    for b in range(bands):
        x = x_refs[b][0]
        t = t_ref[pl.ds(b * band_rows + i * r, r), :]
        rank = _band_rank(x, t, nchunks)
        a1 = a1 + jnp.sum((rank < 1.0).astype(jnp.float32)).reshape(1, 1)
        a5 = a5 + jnp.sum((rank < 5.0).astype(jnp.float32)).reshape(1, 1)
    a1 = a1 * scale
    a5 = a5 * scale

    @pl.when(i == 0)
    def _init():
        acc1_ref[...] = a1
        acc5_ref[...] = a5

    @pl.when(i != 0)
    def _accum():
        acc1_ref[...] += a1
        acc5_ref[...] += a5


@jax.jit
def kernel(outputs, targets):
    b, n = outputs.shape
    bands = _BANDS
    band_rows = b // bands
    r = _ROWS_PER_STEP
    xr = outputs.reshape(bands, band_rows, n)
    t2 = targets.astype(jnp.int32).reshape(b, 1)
    body = functools.partial(_acc_kernel, scale=100.0 / b, nchunks=_NCHUNKS,
                             bands=bands, band_rows=band_rows)

    def make_spec(band):
        return pl.BlockSpec((1, r, n), lambda i, bb=band: (bb, i, 0))

    a1, a5 = pl.pallas_call(
        body,
        grid=(band_rows // r,),
        in_specs=[make_spec(band) for band in range(bands)] + [
            pl.BlockSpec((b, 1), lambda i: (0, 0)),
        ],
        out_specs=[
            pl.BlockSpec((1, 1), lambda i: (0, 0)),
            pl.BlockSpec((1, 1), lambda i: (0, 0)),
        ],
        out_shape=[
            jax.ShapeDtypeStruct((1, 1), jnp.float32),
            jax.ShapeDtypeStruct((1, 1), jnp.float32),
        ],
    )(*([xr] * bands + [t2]))
    return (a1.reshape(1), a5.reshape(1))
